# baseline (speedup 1.0000x reference)
import jax
import jax.numpy as jnp
from jax import lax
from jax.experimental import pallas as pl
from jax.experimental.pallas import tpu as pltpu
from jax.experimental.pallas import tpu_sc as plsc

_N = 16384
_D = 64
_NW = 32
_COLS_PER_W = _N // _NW
_H = _COLS_PER_W // 2
_CHUNKS = _H // 16


def _sc_body(x_hbm, out_hbm, gu0, gi0, gu1, gi1, out_v, sem0, sem1):
    wid = lax.axis_index("s") * 2 + lax.axis_index("c")
    c0 = wid * _COLS_PER_W

    cp0a = pltpu.make_async_copy(x_hbm.at[0, :, pl.ds(c0, _H)], gu0, sem0)
    cp0b = pltpu.make_async_copy(x_hbm.at[1, :, pl.ds(c0, _H)], gi0, sem0)
    cp1a = pltpu.make_async_copy(x_hbm.at[0, :, pl.ds(c0 + _H, _H)], gu1, sem1)
    cp1b = pltpu.make_async_copy(x_hbm.at[1, :, pl.ds(c0 + _H, _H)], gi1, sem1)
    cp0a.start()
    cp0b.start()
    cp1a.start()
    cp1b.start()

    def make_half(a_v, b_v, out_base):
        def chunk(s, carry):
            off = s * 16
            acc = jnp.zeros((16,), jnp.float32)
            for k in range(_D):
                acc = acc + a_v[k, pl.ds(off, 16)] * b_v[k, pl.ds(off, 16)]
            out_v[pl.ds(out_base + off, 16)] = acc
            return carry
        return chunk

    cp0a.wait()
    cp0b.wait()
    lax.fori_loop(0, _CHUNKS, make_half(gu0, gi0, 0), 0)
    cp1a.wait()
    cp1b.wait()
    lax.fori_loop(0, _CHUNKS, make_half(gu1, gi1, _H), 0)
    pltpu.sync_copy(out_v, out_hbm.at[pl.ds(c0, _COLS_PER_W)])


def _sc_rowdot(x):
    mesh = plsc.VectorSubcoreMesh(core_axis_name="c", subcore_axis_name="s")
    return pl.kernel(
        _sc_body,
        mesh=mesh,
        out_type=jax.ShapeDtypeStruct((_N,), jnp.float32),
        scratch_types=[
            pltpu.VMEM((_D, _H), jnp.float32),
            pltpu.VMEM((_D, _H), jnp.float32),
            pltpu.VMEM((_D, _H), jnp.float32),
            pltpu.VMEM((_D, _H), jnp.float32),
            pltpu.VMEM((_COLS_PER_W,), jnp.float32),
            pltpu.SemaphoreType.DMA,
            pltpu.SemaphoreType.DMA,
        ],
    )(x)


def kernel(inputs):
    t = jnp.swapaxes(inputs, 1, 2)
    return _sc_rowdot(t)


# speedup vs baseline: 6.2863x; 6.2863x over previous
import jax
import jax.numpy as jnp
from jax.experimental import pallas as pl
from jax.experimental.pallas import tpu as pltpu

_Q = 4


def _rowdot_kernel(x_hbm, out_ref, *rest):
    bufs = rest[: 2 * _Q]
    sems = rest[2 * _Q :]
    n = out_ref.shape[0]
    qcols = n // _Q
    copies = []
    for q in range(_Q):
        a = pltpu.make_async_copy(
            x_hbm.at[0, :, pl.ds(q * qcols, qcols)], bufs[2 * q], sems[q]
        )
        b = pltpu.make_async_copy(
            x_hbm.at[1, :, pl.ds(q * qcols, qcols)], bufs[2 * q + 1], sems[q]
        )
        a.start()
        b.start()
        copies.append((a, b))
    for q in range(_Q):
        a, b = copies[q]
        a.wait()
        b.wait()
        out_ref[pl.ds(q * qcols, qcols)] = jnp.sum(
            bufs[2 * q][...] * bufs[2 * q + 1][...], axis=0
        )


def kernel(inputs):
    n = inputs.shape[1]
    d = inputs.shape[2]
    t = jnp.swapaxes(inputs, 1, 2)
    qcols = n // _Q
    return pl.pallas_call(
        _rowdot_kernel,
        in_specs=[pl.BlockSpec(memory_space=pltpu.MemorySpace.HBM)],
        out_specs=pl.BlockSpec(memory_space=pltpu.MemorySpace.VMEM),
        out_shape=jax.ShapeDtypeStruct((n,), inputs.dtype),
        scratch_shapes=(
            [pltpu.VMEM((d, qcols), jnp.float32) for _ in range(2 * _Q)]
            + [pltpu.SemaphoreType.DMA for _ in range(_Q)]
        ),
    )(t)
